# baseline (speedup 1.0000x reference)
import functools

import jax
import jax.numpy as jnp
import numpy as np
from jax import lax
from jax.experimental import pallas as pl
from jax.experimental.pallas import tpu as pltpu
from jax.experimental.pallas import tpu_sc as plsc

RES = 128
NPIX = RES * RES
NV = 6
TRANS = -1.4
EPS = 1e-12
LANES = 16
NUM_WORKERS = 32

_VIEWS = np.asarray([
    [[0 * np.pi / 2, 0, np.pi / 2], [0, 0, TRANS]],
    [[1 * np.pi / 2, 0, np.pi / 2], [0, 0, TRANS]],
    [[2 * np.pi / 2, 0, np.pi / 2], [0, 0, TRANS]],
    [[3 * np.pi / 2, 0, np.pi / 2], [0, 0, TRANS]],
    [[0, -np.pi / 2, np.pi / 2], [0, 0, TRANS]],
    [[0, np.pi / 2, np.pi / 2], [0, 0, TRANS]],
])


def _euler2mat_np(angle):
    x, y, z = angle[:, 0], angle[:, 1], angle[:, 2]
    cosz, sinz = np.cos(z), np.sin(z)
    zero, one = np.zeros_like(z), np.ones_like(z)
    zmat = np.stack([cosz, -sinz, zero, sinz, cosz, zero, zero, zero, one],
                    axis=1).reshape(-1, 3, 3)
    cosy, siny = np.cos(y), np.sin(y)
    ymat = np.stack([cosy, zero, siny, zero, one, zero, -siny, zero, cosy],
                    axis=1).reshape(-1, 3, 3)
    cosx, sinx = np.cos(x), np.sin(x)
    xmat = np.stack([one, zero, zero, zero, cosx, -sinx, zero, sinx, cosx],
                    axis=1).reshape(-1, 3, 3)
    return xmat @ ymat @ zmat


import ml_dtypes

_ANGLE_F32 = _VIEWS[:, 0, :].astype(np.float32)
_RM = np.transpose(
    _euler2mat_np(_ANGLE_F32.astype(np.float64)).astype(np.float32),
    (0, 2, 1)).astype(ml_dtypes.bfloat16).astype(np.float32)
_TR = _VIEWS[:, 1, :].astype(np.float32)


def _bf16_round(x):
    i = plsc.bitcast(x, jnp.int32)
    r = (i + 32767 + ((i >> 16) & 1)) & (-65536)
    return plsc.bitcast(r, jnp.float32)


def _ceil_i32(t):
    tc = jnp.minimum(jnp.maximum(t, -4.0), 200.0)
    it = tc.astype(jnp.int32)
    return it + (tc > it.astype(jnp.float32)).astype(jnp.int32)


def _make_sc_call(batch, npts):
    assert batch % NUM_WORKERS == 0
    b_per_w = batch // NUM_WORKERS
    mesh = plsc.VectorSubcoreMesh(core_axis_name="c", subcore_axis_name="s",
                                  num_cores=2, num_subcores=16)

    @functools.partial(
        pl.kernel,
        mesh=mesh,
        compiler_params=pltpu.CompilerParams(needs_layout_passes=False),
        out_type=jax.ShapeDtypeStruct((batch * NV, RES, RES), jnp.float32),
        scratch_types=[
            pltpu.VMEM((npts * 3,), jnp.float32),
            pltpu.VMEM((3, npts), jnp.float32),
            pltpu.VMEM((NPIX,), jnp.float32),
            pltpu.VMEM((NPIX,), jnp.float32),
            pltpu.VMEM((RES, RES), jnp.float32),
            pltpu.VMEM((RES, RES), jnp.float32),
            pltpu.SemaphoreType.DMA,
            pltpu.SemaphoreType.DMA,
        ],
    )
    def sc_call(pts_hbm, out_hbm, pts_raw, pts_v, ws_v, wvs_v, out_v0,
                out_v1, sem0, sem1):
        wid = lax.axis_index("s") * 2 + lax.axis_index("c")

        @plsc.parallel_loop(0, NPIX, step=LANES, unroll=4)
        def _zero(o):
            zz = jnp.zeros((LANES,), jnp.float32)
            ws_v[pl.ds(o, LANES)] = zz
            wvs_v[pl.ds(o, LANES)] = zz

        out_bufs = (out_v0, out_v1)
        sems = (sem0, sem1)

        def batch_body(bloc, carry):
            b = wid * b_per_w + bloc
            pltpu.sync_copy(pts_hbm.at[pl.ds(b * npts * 3, npts * 3)],
                            pts_raw)

            @plsc.parallel_loop(0, npts, step=LANES, unroll=2)
            def _tr(o):
                base = (lax.iota(jnp.int32, LANES) + o) * 3
                for k in range(3):
                    x = plsc.load_gather(pts_raw, [base + k])
                    pts_v[k, pl.ds(o, LANES)] = _bf16_round(x)

            for v in range(NV):
                m = _RM[v]
                t2 = float(_TR[v, 2])
                out_v = out_bufs[v % 2]
                sem = sems[v % 2]

                @plsc.parallel_loop(0, npts, step=LANES, unroll=4)
                def _pts(o, m=m, t2=t2):
                    s = pl.ds(o, LANES)
                    px = pts_v[0, s]
                    py = pts_v[1, s]
                    pz = pts_v[2, s]
                    X = (px * float(m[0, 0])
                         + (py * float(m[1, 0]) + pz * float(m[2, 0])))
                    Y = (px * float(m[0, 1])
                         + (py * float(m[1, 1]) + pz * float(m[2, 1])))
                    Z = (px * float(m[0, 2])
                         + (py * float(m[1, 2]) + pz * float(m[2, 2]))) - t2
                    zp = Z + EPS
                    w = 1.0 / zp
                    fx = (X / zp + 1.0) * 64.0
                    fy = (Y / zp + 1.0) * 64.0
                    ix = _ceil_i32(fx - 0.5)
                    iy = _ceil_i32(fy - 0.5)
                    ok = ((ix >= 0) & (ix <= RES - 1) & (iy >= 0)
                          & (iy <= RES - 1) & (Z >= 0.0))
                    ixc = jnp.minimum(jnp.maximum(ix, 0), RES - 1)
                    iyc = jnp.minimum(jnp.maximum(iy, 0), RES - 1)
                    idx = ixc * RES + iyc
                    plsc.addupdate_scatter(ws_v, [idx], w, mask=ok)
                    plsc.addupdate_scatter(wvs_v, [idx], Z * w, mask=ok)

                if v >= 2:
                    pltpu.make_async_copy(out_v, out_hbm.at[0], sem).wait()
                else:
                    @pl.when(bloc > 0)
                    def _(out_v=out_v, sem=sem):
                        pltpu.make_async_copy(out_v, out_hbm.at[0],
                                              sem).wait()

                @plsc.parallel_loop(0, RES, step=1, unroll=2)
                def _fin(r, out_v=out_v):
                    for cc in range(RES // LANES):
                        s = pl.ds(r * RES + cc * LANES, LANES)
                        wsum = ws_v[s]
                        den = jnp.where(wsum == 0.0, 1.0, wsum)
                        out_v[r, pl.ds(cc * LANES, LANES)] = wvs_v[s] / den
                        zz = jnp.zeros((LANES,), jnp.float32)
                        ws_v[s] = zz
                        wvs_v[s] = zz

                pltpu.make_async_copy(out_v, out_hbm.at[b * NV + v],
                                      sem).start()
            return carry

        lax.fori_loop(0, b_per_w, batch_body, 0)
        pltpu.make_async_copy(out_v0, out_hbm.at[0], sem0).wait()
        pltpu.make_async_copy(out_v1, out_hbm.at[0], sem1).wait()

    return sc_call


def kernel(points):
    batch, npts, _ = points.shape
    return _make_sc_call(batch, npts)(points.reshape(batch * npts * 3))


# speedup vs baseline: 10.9160x; 10.9160x over previous
import functools

import jax
import jax.numpy as jnp
import numpy as np
from jax import lax
from jax.experimental import pallas as pl
from jax.experimental.pallas import tpu as pltpu
from jax.experimental.pallas import tpu_sc as plsc

RES = 128
NPIX = RES * RES
NV = 6
TRANS = -1.4
EPS = 1e-12
LANES = 16
NUM_WORKERS = 32

_VIEWS = np.asarray([
    [[0 * np.pi / 2, 0, np.pi / 2], [0, 0, TRANS]],
    [[1 * np.pi / 2, 0, np.pi / 2], [0, 0, TRANS]],
    [[2 * np.pi / 2, 0, np.pi / 2], [0, 0, TRANS]],
    [[3 * np.pi / 2, 0, np.pi / 2], [0, 0, TRANS]],
    [[0, -np.pi / 2, np.pi / 2], [0, 0, TRANS]],
    [[0, np.pi / 2, np.pi / 2], [0, 0, TRANS]],
])


def _euler2mat_np(angle):
    x, y, z = angle[:, 0], angle[:, 1], angle[:, 2]
    cosz, sinz = np.cos(z), np.sin(z)
    zero, one = np.zeros_like(z), np.ones_like(z)
    zmat = np.stack([cosz, -sinz, zero, sinz, cosz, zero, zero, zero, one],
                    axis=1).reshape(-1, 3, 3)
    cosy, siny = np.cos(y), np.sin(y)
    ymat = np.stack([cosy, zero, siny, zero, one, zero, -siny, zero, cosy],
                    axis=1).reshape(-1, 3, 3)
    cosx, sinx = np.cos(x), np.sin(x)
    xmat = np.stack([one, zero, zero, zero, cosx, -sinx, zero, sinx, cosx],
                    axis=1).reshape(-1, 3, 3)
    return xmat @ ymat @ zmat


import ml_dtypes

_ANGLE_F32 = _VIEWS[:, 0, :].astype(np.float32)
_RM = np.transpose(
    _euler2mat_np(_ANGLE_F32.astype(np.float64)).astype(np.float32),
    (0, 2, 1)).astype(ml_dtypes.bfloat16).astype(np.float32)
_TR = _VIEWS[:, 1, :].astype(np.float32)


def _bf16_round(x):
    i = plsc.bitcast(x, jnp.int32)
    r = (i + 32767 + ((i >> 16) & 1)) & (-65536)
    return plsc.bitcast(r, jnp.float32)


def _ceil_i32(t):
    tc = jnp.minimum(jnp.maximum(t, -4.0), 200.0)
    it = tc.astype(jnp.int32)
    return it + (tc > it.astype(jnp.float32)).astype(jnp.int32)


def _make_sc_call(batch, npts):
    assert batch % NUM_WORKERS == 0
    b_per_w = batch // NUM_WORKERS
    mesh = plsc.VectorSubcoreMesh(core_axis_name="c", subcore_axis_name="s",
                                  num_cores=2, num_subcores=16)

    @functools.partial(
        pl.kernel,
        mesh=mesh,
        compiler_params=pltpu.CompilerParams(needs_layout_passes=False),
        out_type=jax.ShapeDtypeStruct((batch * NV, RES, RES), jnp.float32),
        scratch_types=[
            pltpu.VMEM((npts * 3,), jnp.float32),
            pltpu.VMEM((3, npts), jnp.float32),
            pltpu.VMEM((NPIX,), jnp.float32),
            pltpu.VMEM((NPIX,), jnp.float32),
            pltpu.VMEM((RES, RES), jnp.float32),
            pltpu.VMEM((RES, RES), jnp.float32),
            pltpu.SemaphoreType.DMA,
            pltpu.SemaphoreType.DMA,
        ],
    )
    def sc_call(pts_hbm, out_hbm, pts_raw, pts_v, ws_v, wvs_v, out_v0,
                out_v1, sem0, sem1):
        wid = lax.axis_index("s") * 2 + lax.axis_index("c")

        @plsc.parallel_loop(0, NPIX, step=LANES, unroll=4)
        def _zero(o):
            zz = jnp.zeros((LANES,), jnp.float32)
            ws_v[pl.ds(o, LANES)] = zz
            wvs_v[pl.ds(o, LANES)] = zz

        out_bufs = (out_v0, out_v1)
        sems = (sem0, sem1)

        def batch_body(bloc, carry):
            b = wid * b_per_w + bloc
            pltpu.sync_copy(pts_hbm.at[b], pts_raw)

            @plsc.parallel_loop(0, npts, step=LANES, unroll=2)
            def _tr(o):
                base = (lax.iota(jnp.int32, LANES) + o) * 3
                for k in range(3):
                    x = plsc.load_gather(pts_raw, [base + k])
                    pts_v[k, pl.ds(o, LANES)] = _bf16_round(x)

            for v in range(NV):
                m = _RM[v]
                t2 = float(_TR[v, 2])
                out_v = out_bufs[v % 2]
                sem = sems[v % 2]

                @plsc.parallel_loop(0, npts, step=LANES, unroll=4)
                def _pts(o, m=m, t2=t2):
                    s = pl.ds(o, LANES)
                    px = pts_v[0, s]
                    py = pts_v[1, s]
                    pz = pts_v[2, s]
                    X = (px * float(m[0, 0])
                         + (py * float(m[1, 0]) + pz * float(m[2, 0])))
                    Y = (px * float(m[0, 1])
                         + (py * float(m[1, 1]) + pz * float(m[2, 1])))
                    Z = (px * float(m[0, 2])
                         + (py * float(m[1, 2]) + pz * float(m[2, 2]))) - t2
                    zp = Z + EPS
                    w = 1.0 / zp
                    fx = (X / zp + 1.0) * 64.0
                    fy = (Y / zp + 1.0) * 64.0
                    ix = _ceil_i32(fx - 0.5)
                    iy = _ceil_i32(fy - 0.5)
                    ok = ((ix >= 0) & (ix <= RES - 1) & (iy >= 0)
                          & (iy <= RES - 1) & (Z >= 0.0))
                    ixc = jnp.minimum(jnp.maximum(ix, 0), RES - 1)
                    iyc = jnp.minimum(jnp.maximum(iy, 0), RES - 1)
                    idx = ixc * RES + iyc
                    plsc.addupdate_scatter(ws_v, [idx], w, mask=ok)
                    plsc.addupdate_scatter(wvs_v, [idx], Z * w, mask=ok)

                if v >= 2:
                    pltpu.make_async_copy(out_v, out_hbm.at[0], sem).wait()
                else:
                    @pl.when(bloc > 0)
                    def _(out_v=out_v, sem=sem):
                        pltpu.make_async_copy(out_v, out_hbm.at[0],
                                              sem).wait()

                @plsc.parallel_loop(0, RES, step=1, unroll=2)
                def _fin(r, out_v=out_v):
                    for cc in range(RES // LANES):
                        s = pl.ds(r * RES + cc * LANES, LANES)
                        wsum = ws_v[s]
                        den = jnp.where(wsum == 0.0, 1.0, wsum)
                        out_v[r, pl.ds(cc * LANES, LANES)] = wvs_v[s] / den
                        zz = jnp.zeros((LANES,), jnp.float32)
                        ws_v[s] = zz
                        wvs_v[s] = zz

                pltpu.make_async_copy(out_v, out_hbm.at[b * NV + v],
                                      sem).start()
            return carry

        lax.fori_loop(0, b_per_w, batch_body, 0)
        pltpu.make_async_copy(out_v0, out_hbm.at[0], sem0).wait()
        pltpu.make_async_copy(out_v1, out_hbm.at[0], sem1).wait()

    return sc_call


def kernel(points):
    batch, npts, _ = points.shape
    return _make_sc_call(batch, npts)(points.reshape(batch, npts * 3))


# speedup vs baseline: 11.7802x; 1.0792x over previous
import functools

import jax
import jax.numpy as jnp
import numpy as np
from jax import lax
from jax.experimental import pallas as pl
from jax.experimental.pallas import tpu as pltpu
from jax.experimental.pallas import tpu_sc as plsc

RES = 128
NPIX = RES * RES
NV = 6
TRANS = -1.4
EPS = 1e-12
LANES = 16
NUM_WORKERS = 32

_VIEWS = np.asarray([
    [[0 * np.pi / 2, 0, np.pi / 2], [0, 0, TRANS]],
    [[1 * np.pi / 2, 0, np.pi / 2], [0, 0, TRANS]],
    [[2 * np.pi / 2, 0, np.pi / 2], [0, 0, TRANS]],
    [[3 * np.pi / 2, 0, np.pi / 2], [0, 0, TRANS]],
    [[0, -np.pi / 2, np.pi / 2], [0, 0, TRANS]],
    [[0, np.pi / 2, np.pi / 2], [0, 0, TRANS]],
])


def _euler2mat_np(angle):
    x, y, z = angle[:, 0], angle[:, 1], angle[:, 2]
    cosz, sinz = np.cos(z), np.sin(z)
    zero, one = np.zeros_like(z), np.ones_like(z)
    zmat = np.stack([cosz, -sinz, zero, sinz, cosz, zero, zero, zero, one],
                    axis=1).reshape(-1, 3, 3)
    cosy, siny = np.cos(y), np.sin(y)
    ymat = np.stack([cosy, zero, siny, zero, one, zero, -siny, zero, cosy],
                    axis=1).reshape(-1, 3, 3)
    cosx, sinx = np.cos(x), np.sin(x)
    xmat = np.stack([one, zero, zero, zero, cosx, -sinx, zero, sinx, cosx],
                    axis=1).reshape(-1, 3, 3)
    return xmat @ ymat @ zmat


import ml_dtypes

_ANGLE_F32 = _VIEWS[:, 0, :].astype(np.float32)
_RM = np.transpose(
    _euler2mat_np(_ANGLE_F32.astype(np.float64)).astype(np.float32),
    (0, 2, 1)).astype(ml_dtypes.bfloat16).astype(np.float32)
_TR = _VIEWS[:, 1, :].astype(np.float32)


def _bf16_round(x):
    i = plsc.bitcast(x, jnp.int32)
    r = (i + 32767 + ((i >> 16) & 1)) & (-65536)
    return plsc.bitcast(r, jnp.float32)


def _ceil_i32(t):
    tc = jnp.minimum(jnp.maximum(t, -4.0), 200.0)
    it = tc.astype(jnp.int32)
    return it + (tc > it.astype(jnp.float32)).astype(jnp.int32)


def _make_sc_call(batch, npts):
    assert batch % NUM_WORKERS == 0
    b_per_w = batch // NUM_WORKERS
    mesh = plsc.VectorSubcoreMesh(core_axis_name="c", subcore_axis_name="s",
                                  num_cores=2, num_subcores=16)

    @functools.partial(
        pl.kernel,
        mesh=mesh,
        compiler_params=pltpu.CompilerParams(needs_layout_passes=False),
        out_type=jax.ShapeDtypeStruct((batch * NV, RES, RES), jnp.float32),
        scratch_types=[
            pltpu.VMEM((3, npts), jnp.float32),
            pltpu.VMEM((NPIX,), jnp.float32),
            pltpu.VMEM((NPIX,), jnp.float32),
            pltpu.VMEM((RES, RES), jnp.float32),
            pltpu.VMEM((RES, RES), jnp.float32),
            pltpu.SemaphoreType.DMA,
            pltpu.SemaphoreType.DMA,
        ],
    )
    def sc_call(pts_hbm, out_hbm, pts_v, ws_v, wvs_v, out_v0,
                out_v1, sem0, sem1):
        wid = lax.axis_index("s") * 2 + lax.axis_index("c")

        @plsc.parallel_loop(0, NPIX, step=LANES, unroll=4)
        def _zero(o):
            zz = jnp.zeros((LANES,), jnp.float32)
            ws_v[pl.ds(o, LANES)] = zz
            wvs_v[pl.ds(o, LANES)] = zz

        out_bufs = (out_v0, out_v1)
        sems = (sem0, sem1)

        def batch_body(bloc, carry):
            b = wid * b_per_w + bloc
            pltpu.sync_copy(pts_hbm.at[b], pts_v)
            for v in range(NV):
                m = _RM[v]
                t2 = float(_TR[v, 2])
                out_v = out_bufs[v % 2]
                sem = sems[v % 2]

                @plsc.parallel_loop(0, npts, step=LANES, unroll=4)
                def _pts(o, m=m, t2=t2):
                    s = pl.ds(o, LANES)
                    px = pts_v[0, s]
                    py = pts_v[1, s]
                    pz = pts_v[2, s]
                    X = (px * float(m[0, 0])
                         + (py * float(m[1, 0]) + pz * float(m[2, 0])))
                    Y = (px * float(m[0, 1])
                         + (py * float(m[1, 1]) + pz * float(m[2, 1])))
                    Z = (px * float(m[0, 2])
                         + (py * float(m[1, 2]) + pz * float(m[2, 2]))) - t2
                    zp = Z + EPS
                    w = 1.0 / zp
                    fx = (X / zp + 1.0) * 64.0
                    fy = (Y / zp + 1.0) * 64.0
                    ix = _ceil_i32(fx - 0.5)
                    iy = _ceil_i32(fy - 0.5)
                    ok = ((ix >= 0) & (ix <= RES - 1) & (iy >= 0)
                          & (iy <= RES - 1) & (Z >= 0.0))
                    ixc = jnp.minimum(jnp.maximum(ix, 0), RES - 1)
                    iyc = jnp.minimum(jnp.maximum(iy, 0), RES - 1)
                    idx = ixc * RES + iyc
                    plsc.addupdate_scatter(ws_v, [idx], w, mask=ok)
                    plsc.addupdate_scatter(wvs_v, [idx], Z * w, mask=ok)

                if v >= 2:
                    pltpu.make_async_copy(out_v, out_hbm.at[0], sem).wait()
                else:
                    @pl.when(bloc > 0)
                    def _(out_v=out_v, sem=sem):
                        pltpu.make_async_copy(out_v, out_hbm.at[0],
                                              sem).wait()

                @plsc.parallel_loop(0, RES, step=1, unroll=2)
                def _fin(r, out_v=out_v):
                    for cc in range(RES // LANES):
                        s = pl.ds(r * RES + cc * LANES, LANES)
                        wsum = ws_v[s]
                        den = jnp.where(wsum == 0.0, 1.0, wsum)
                        out_v[r, pl.ds(cc * LANES, LANES)] = wvs_v[s] / den
                        zz = jnp.zeros((LANES,), jnp.float32)
                        ws_v[s] = zz
                        wvs_v[s] = zz

                pltpu.make_async_copy(out_v, out_hbm.at[b * NV + v],
                                      sem).start()
            return carry

        lax.fori_loop(0, b_per_w, batch_body, 0)
        pltpu.make_async_copy(out_v0, out_hbm.at[0], sem0).wait()
        pltpu.make_async_copy(out_v1, out_hbm.at[0], sem1).wait()

    return sc_call


def kernel(points):
    batch, npts, _ = points.shape
    points = lax.reduce_precision(points, exponent_bits=8, mantissa_bits=7)
    pts_t = jnp.transpose(points, (0, 2, 1))
    return _make_sc_call(batch, npts)(pts_t)
